# baseline (speedup 1.0000x reference)
import functools

import jax
import jax.numpy as jnp
from jax import lax
from jax.experimental import pallas as pl
from jax.experimental.pallas import tpu as pltpu
from jax.experimental.pallas import tpu_sc as plsc

N = 10000
E = 320000
D = 128
R = 32
NB = 8

NC = 2
NS = 16

K = 64
CB = 40
BPW = 160
NCHUNK = BPW // CB
BPAIR = NC * BPW
NROWS = NS * BPAIR
E_PAD = NROWS * K
NPAD = 10112
ROWS_PER_TILE = NPAD // NS
NSLOT = 4


def _tc_xw_body(x_ref, wc_ref, bases_ref, out_ref):
    w = wc_ref[0, 0, 0] * bases_ref[0]
    for b in range(1, NB):
        w += wc_ref[0, 0, b] * bases_ref[b]
    out_ref[0] = jnp.dot(x_ref[...], w, preferred_element_type=jnp.float32)


def _tc_xw(x, w_comp, weight_bases):
    return pl.pallas_call(
        _tc_xw_body,
        grid=(R,),
        in_specs=[
            pl.BlockSpec((N, D), lambda r: (0, 0)),
            pl.BlockSpec((1, 1, NB), lambda r: (r, 0, 0)),
            pl.BlockSpec((NB, D, D), lambda r: (0, 0, 0)),
        ],
        out_specs=pl.BlockSpec((1, N, D), lambda r: (r, 0, 0)),
        out_shape=jax.ShapeDtypeStruct((R, N, D), jnp.float32),
    )(x, w_comp.reshape(R, 1, NB), weight_bases)


def _sc_agg_body(e3_hbm, xw_hbm, out_hbm,
                 srcb, relb, dstb,
                 rows0, rows1, rows2, rows3, agg,
                 sg0, sg1, sg2, sg3, ss0, ss1, ss2, ss3):
    c = lax.axis_index("c")
    s = lax.axis_index("s")

    @pl.loop(0, K)
    def _(i):
        for j in range(D // 16):
            rows0[i, pl.ds(j * 16, 16)] = jnp.zeros((16,), jnp.float32)

    for k in range(ROWS_PER_TILE // K + 1):
        cnt = K if k < ROWS_PER_TILE // K else ROWS_PER_TILE % K
        pltpu.sync_copy(rows0.at[pl.ds(0, cnt)],
                        agg.at[pl.ds(s * ROWS_PER_TILE + k * K, cnt)])
    plsc.subcore_barrier()

    rows = (rows0, rows1, rows2, rows3)
    sg = (sg0, sg1, sg2, sg3)
    ss = (ss0, ss1, ss2, ss3)
    rowbase = (s * NC + c) * BPW

    @pl.loop(0, NCHUNK)
    def _(cc):
        csl = pl.ds(rowbase + cc * CB, CB)
        pltpu.sync_copy(e3_hbm.at[0, csl], srcb)
        pltpu.sync_copy(e3_hbm.at[2, csl], relb)
        pltpu.sync_copy(e3_hbm.at[1, csl], dstb)

        for i in range(CB):
            for j in range(K // 16):
                sl = pl.ds(j * 16, 16)
                srcb[i, sl] = relb[i, sl] * N + srcb[i, sl]

        pltpu.async_copy(xw_hbm.at[srcb.at[0]], rows[0], sg[0])
        pltpu.async_copy(xw_hbm.at[srcb.at[1]], rows[1], sg[1])
        for i in range(CB):
            sl = i % NSLOT
            pltpu.make_async_copy(
                xw_hbm.at[srcb.at[i]], rows[sl], sg[sl]).wait()
            pltpu.async_copy(rows[sl], agg.at[dstb.at[i]], ss[sl], add=True)
            if i >= 2:
                psl = (i - 2) % NSLOT
                pltpu.make_async_copy(
                    rows[psl], agg.at[dstb.at[i - 2]], ss[psl]).wait()
            if i + 2 < CB:
                nsl = (i + 2) % NSLOT
                pltpu.async_copy(
                    xw_hbm.at[srcb.at[i + 2]], rows[nsl], sg[nsl])
        for i in range(CB - 2, CB):
            sl = i % NSLOT
            pltpu.make_async_copy(
                rows[sl], agg.at[dstb.at[i]], ss[sl]).wait()

    plsc.subcore_barrier()
    pltpu.sync_copy(agg.at[pl.ds(s * ROWS_PER_TILE, ROWS_PER_TILE)],
                    out_hbm.at[c, pl.ds(s * ROWS_PER_TILE, ROWS_PER_TILE)])


def _sc_agg(e3, xw_flat):
    mesh = plsc.VectorSubcoreMesh(core_axis_name="c", subcore_axis_name="s")
    return pl.kernel(
        _sc_agg_body,
        out_type=jax.ShapeDtypeStruct((NC, NPAD, D), jnp.float32),
        mesh=mesh,
        scratch_types=[
            pltpu.VMEM((CB, K), jnp.int32),
            pltpu.VMEM((CB, K), jnp.int32),
            pltpu.VMEM((CB, K), jnp.int32),
            pltpu.VMEM((K, D), jnp.float32),
            pltpu.VMEM((K, D), jnp.float32),
            pltpu.VMEM((K, D), jnp.float32),
            pltpu.VMEM((K, D), jnp.float32),
            pltpu.VMEM_SHARED((NPAD, D), jnp.float32),
            pltpu.SemaphoreType.DMA,
            pltpu.SemaphoreType.DMA,
            pltpu.SemaphoreType.DMA,
            pltpu.SemaphoreType.DMA,
            pltpu.SemaphoreType.DMA,
            pltpu.SemaphoreType.DMA,
            pltpu.SemaphoreType.DMA,
            pltpu.SemaphoreType.DMA,
        ],
    )(e3, xw_flat)


def _tc_curr_body(x_ref, w_ref, b_ref, out_ref):
    acc = jnp.dot(x_ref[...], w_ref[...], preferred_element_type=jnp.float32)
    out_ref[...] = acc + b_ref[...]


def _tc_curr(x, self_loop_weight, bias):
    blk = 1000
    return pl.pallas_call(
        _tc_curr_body,
        grid=(N // blk,),
        in_specs=[
            pl.BlockSpec((blk, D), lambda i: (i, 0)),
            pl.BlockSpec((D, D), lambda i: (0, 0)),
            pl.BlockSpec((1, D), lambda i: (0, 0)),
        ],
        out_specs=pl.BlockSpec((blk, D), lambda i: (i, 0)),
        out_shape=jax.ShapeDtypeStruct((N, D), jnp.float32),
    )(x, self_loop_weight, bias.reshape(1, D))


def _tc_out_body(curr_ref, agg_ref, out_ref):
    out_ref[...] = jnp.maximum(curr_ref[...] + agg_ref[0] + agg_ref[1], 0.0)


def _tc_out(curr, agg_pair):
    blk = 1000
    return pl.pallas_call(
        _tc_out_body,
        grid=(N // blk,),
        in_specs=[
            pl.BlockSpec((blk, D), lambda i: (i, 0)),
            pl.BlockSpec((NC, blk, D), lambda i: (0, i, 0)),
        ],
        out_specs=pl.BlockSpec((blk, D), lambda i: (i, 0)),
        out_shape=jax.ShapeDtypeStruct((N, D), jnp.float32),
    )(curr, agg_pair)


def kernel(x, edge_index, edge_type, weight_bases, w_comp, self_loop_weight, bias):
    pad = E_PAD - E
    ar = jnp.arange(pad, dtype=jnp.int32)
    pad_block = jnp.stack([ar % N, N + (ar % (NPAD - N)), jnp.zeros((pad,), jnp.int32)])
    e3 = jnp.concatenate(
        [jnp.concatenate([edge_index.astype(jnp.int32),
                          edge_type[None].astype(jnp.int32)], axis=0),
         pad_block], axis=1).reshape(3, NROWS, K)

    xw = _tc_xw(x, w_comp, weight_bases)
    curr = _tc_curr(x, self_loop_weight, bias)
    agg_pair = _sc_agg(e3, xw.reshape(R * N, D))
    return _tc_out(curr, agg_pair)


# speedup vs baseline: 1.0597x; 1.0597x over previous
import functools

import jax
import jax.numpy as jnp
from jax import lax
from jax.experimental import pallas as pl
from jax.experimental.pallas import tpu as pltpu
from jax.experimental.pallas import tpu_sc as plsc

N = 10000
E = 320000
D = 128
R = 32
NB = 8

NC = 2
NS = 16

K = 64
CB = 40
BPW = 160
NCHUNK = BPW // CB
NROWS = E // K
NPAD = 10112
ROWS_PER_TILE = NPAD // NS
NSLOT = 4


def _tc_xw_body(x_ref, wc_ref, bases_ref, out_ref):
    w = wc_ref[0, 0, 0] * bases_ref[0]
    for b in range(1, NB):
        w += wc_ref[0, 0, b] * bases_ref[b]
    out_ref[0] = jnp.dot(x_ref[...], w, preferred_element_type=jnp.float32)


def _tc_xw(x, w_comp, weight_bases):
    return pl.pallas_call(
        _tc_xw_body,
        grid=(R,),
        in_specs=[
            pl.BlockSpec((N, D), lambda r: (0, 0)),
            pl.BlockSpec((1, 1, NB), lambda r: (r, 0, 0)),
            pl.BlockSpec((NB, D, D), lambda r: (0, 0, 0)),
        ],
        out_specs=pl.BlockSpec((1, N, D), lambda r: (r, 0, 0)),
        out_shape=jax.ShapeDtypeStruct((R, N, D), jnp.float32),
    )(x, w_comp.reshape(R, 1, NB), weight_bases)


def _sc_agg_body(ei_hbm, et_hbm, xw_hbm, out_hbm,
                 srcb, relb, dstb,
                 rows0, rows1, rows2, rows3, agg,
                 sg0, sg1, sg2, sg3, ss0, ss1, ss2, ss3):
    c = lax.axis_index("c")
    s = lax.axis_index("s")

    @pl.loop(0, K)
    def _(i):
        for j in range(D // 16):
            rows0[i, pl.ds(j * 16, 16)] = jnp.zeros((16,), jnp.float32)

    for k in range(ROWS_PER_TILE // K + 1):
        cnt = K if k < ROWS_PER_TILE // K else ROWS_PER_TILE % K
        pltpu.sync_copy(rows0.at[pl.ds(0, cnt)],
                        agg.at[pl.ds(s * ROWS_PER_TILE + k * K, cnt)])
    plsc.subcore_barrier()

    rows = (rows0, rows1, rows2, rows3)
    sg = (sg0, sg1, sg2, sg3)
    ss = (ss0, ss1, ss2, ss3)
    wid = s * NC + c
    rowbase = wid * BPW
    nchunk = lax.select(wid == NC * NS - 1, 1, NCHUNK)

    @pl.loop(0, nchunk)
    def _(cc):
        csl = pl.ds(rowbase + cc * CB, CB)
        pltpu.sync_copy(ei_hbm.at[0, csl], srcb)
        pltpu.sync_copy(et_hbm.at[csl], relb)
        pltpu.sync_copy(ei_hbm.at[1, csl], dstb)

        for i in range(CB):
            for j in range(K // 16):
                sl = pl.ds(j * 16, 16)
                srcb[i, sl] = relb[i, sl] * N + srcb[i, sl]

        pltpu.async_copy(xw_hbm.at[srcb.at[0]], rows[0], sg[0])
        pltpu.async_copy(xw_hbm.at[srcb.at[1]], rows[1], sg[1])
        for i in range(CB):
            sl = i % NSLOT
            pltpu.make_async_copy(
                xw_hbm.at[srcb.at[i]], rows[sl], sg[sl]).wait()
            pltpu.async_copy(rows[sl], agg.at[dstb.at[i]], ss[sl], add=True)
            if i >= 2:
                psl = (i - 2) % NSLOT
                pltpu.make_async_copy(
                    rows[psl], agg.at[dstb.at[i - 2]], ss[psl]).wait()
            if i + 2 < CB:
                nsl = (i + 2) % NSLOT
                pltpu.async_copy(
                    xw_hbm.at[srcb.at[i + 2]], rows[nsl], sg[nsl])
        for i in range(CB - 2, CB):
            sl = i % NSLOT
            pltpu.make_async_copy(
                rows[sl], agg.at[dstb.at[i]], ss[sl]).wait()

    plsc.subcore_barrier()
    pltpu.sync_copy(agg.at[pl.ds(s * ROWS_PER_TILE, ROWS_PER_TILE)],
                    out_hbm.at[c, pl.ds(s * ROWS_PER_TILE, ROWS_PER_TILE)])


def _sc_agg(ei, et, xw_flat):
    mesh = plsc.VectorSubcoreMesh(core_axis_name="c", subcore_axis_name="s")
    return pl.kernel(
        _sc_agg_body,
        out_type=jax.ShapeDtypeStruct((NC, NPAD, D), jnp.float32),
        mesh=mesh,
        scratch_types=[
            pltpu.VMEM((CB, K), jnp.int32),
            pltpu.VMEM((CB, K), jnp.int32),
            pltpu.VMEM((CB, K), jnp.int32),
            pltpu.VMEM((K, D), jnp.float32),
            pltpu.VMEM((K, D), jnp.float32),
            pltpu.VMEM((K, D), jnp.float32),
            pltpu.VMEM((K, D), jnp.float32),
            pltpu.VMEM_SHARED((NPAD, D), jnp.float32),
            pltpu.SemaphoreType.DMA,
            pltpu.SemaphoreType.DMA,
            pltpu.SemaphoreType.DMA,
            pltpu.SemaphoreType.DMA,
            pltpu.SemaphoreType.DMA,
            pltpu.SemaphoreType.DMA,
            pltpu.SemaphoreType.DMA,
            pltpu.SemaphoreType.DMA,
        ],
    )(ei, et, xw_flat)


def _tc_curr_body(x_ref, w_ref, b_ref, out_ref):
    acc = jnp.dot(x_ref[...], w_ref[...], preferred_element_type=jnp.float32)
    out_ref[...] = acc + b_ref[...]


def _tc_curr(x, self_loop_weight, bias):
    blk = 1000
    return pl.pallas_call(
        _tc_curr_body,
        grid=(N // blk,),
        in_specs=[
            pl.BlockSpec((blk, D), lambda i: (i, 0)),
            pl.BlockSpec((D, D), lambda i: (0, 0)),
            pl.BlockSpec((1, D), lambda i: (0, 0)),
        ],
        out_specs=pl.BlockSpec((blk, D), lambda i: (i, 0)),
        out_shape=jax.ShapeDtypeStruct((N, D), jnp.float32),
    )(x, self_loop_weight, bias.reshape(1, D))


def _tc_out_body(curr_ref, agg_ref, out_ref):
    out_ref[...] = jnp.maximum(curr_ref[...] + agg_ref[0] + agg_ref[1], 0.0)


def _tc_out(curr, agg_pair):
    blk = 1000
    return pl.pallas_call(
        _tc_out_body,
        grid=(N // blk,),
        in_specs=[
            pl.BlockSpec((blk, D), lambda i: (i, 0)),
            pl.BlockSpec((NC, blk, D), lambda i: (0, i, 0)),
        ],
        out_specs=pl.BlockSpec((blk, D), lambda i: (i, 0)),
        out_shape=jax.ShapeDtypeStruct((N, D), jnp.float32),
    )(curr, agg_pair)


def kernel(x, edge_index, edge_type, weight_bases, w_comp, self_loop_weight, bias):
    ei = edge_index.astype(jnp.int32).reshape(2, NROWS, K)
    et = edge_type.astype(jnp.int32).reshape(NROWS, K)

    xw = _tc_xw(x, w_comp, weight_bases)
    curr = _tc_curr(x, self_loop_weight, bias)
    agg_pair = _sc_agg(ei, et, xw.reshape(R * N, D))
    return _tc_out(curr, agg_pair)


# speedup vs baseline: 1.0772x; 1.0165x over previous
import functools

import jax
import jax.numpy as jnp
from jax import lax
from jax.experimental import pallas as pl
from jax.experimental.pallas import tpu as pltpu
from jax.experimental.pallas import tpu_sc as plsc

N = 10000
E = 320000
D = 128
R = 32
NB = 8

NC = 2
NS = 16

K = 64
CB = 40
BPW = 160
NCHUNK = BPW // CB
NROWS = E // K
NPAD = 10112
ROWS_PER_TILE = NPAD // NS
NSLOT = 4


def _tc_xw_body(x_ref, wc_ref, bases_ref, out_ref):
    w = wc_ref[0, 0, 0] * bases_ref[0]
    for b in range(1, NB):
        w += wc_ref[0, 0, b] * bases_ref[b]
    out_ref[0] = jnp.dot(x_ref[...], w, preferred_element_type=jnp.float32)


def _tc_xw(x, w_comp, weight_bases):
    return pl.pallas_call(
        _tc_xw_body,
        grid=(R,),
        in_specs=[
            pl.BlockSpec((N, D), lambda r: (0, 0)),
            pl.BlockSpec((1, 1, NB), lambda r: (r, 0, 0)),
            pl.BlockSpec((NB, D, D), lambda r: (0, 0, 0)),
        ],
        out_specs=pl.BlockSpec((1, N, D), lambda r: (r, 0, 0)),
        out_shape=jax.ShapeDtypeStruct((R, N, D), jnp.float32),
    )(x, w_comp.reshape(R, 1, NB), weight_bases)


def _sc_agg_body(ef_hbm, et_hbm, xw_hbm, out_hbm,
                 srcb, relb, dstl, dstb,
                 rows0, rows1, rows2, rows3, agg,
                 sg0, sg1, sg2, sg3, ss0, ss1, ss2, ss3):
    c = lax.axis_index("c")
    s = lax.axis_index("s")

    @pl.loop(0, K)
    def _(i):
        for j in range(D // 16):
            rows0[i, pl.ds(j * 16, 16)] = jnp.zeros((16,), jnp.float32)

    for k in range(ROWS_PER_TILE // K + 1):
        cnt = K if k < ROWS_PER_TILE // K else ROWS_PER_TILE % K
        pltpu.sync_copy(rows0.at[pl.ds(0, cnt)],
                        agg.at[pl.ds(s * ROWS_PER_TILE + k * K, cnt)])
    plsc.subcore_barrier()

    rows = (rows0, rows1, rows2, rows3)
    sg = (sg0, sg1, sg2, sg3)
    ss = (ss0, ss1, ss2, ss3)
    wid = s * NC + c
    nchunk = lax.select(wid == NC * NS - 1, 1, NCHUNK)

    @pl.loop(0, nchunk)
    def _(cc):
        ebase = (wid * BPW + cc * CB) * K
        esl = pl.ds(ebase, CB * K)
        pltpu.sync_copy(ef_hbm.at[esl], srcb)
        pltpu.sync_copy(et_hbm.at[esl], relb)
        pltpu.sync_copy(ef_hbm.at[pl.ds(E + ebase, CB * K)], dstl)

        for i in range(CB):
            for j in range(K // 16):
                sl = pl.ds(j * 16, 16)
                fl = pl.ds(i * K + j * 16, 16)
                srcb[fl] = relb[fl] * N + srcb[fl]
                dstb[i, sl] = dstl[fl]

        pltpu.async_copy(xw_hbm.at[srcb.at[pl.ds(0, K)]], rows[0], sg[0])
        pltpu.async_copy(xw_hbm.at[srcb.at[pl.ds(K, K)]], rows[1], sg[1])
        for i in range(CB):
            sl = i % NSLOT
            pltpu.make_async_copy(
                xw_hbm.at[srcb.at[pl.ds(i * K, K)]], rows[sl], sg[sl]).wait()
            pltpu.async_copy(rows[sl], agg.at[dstb.at[i]], ss[sl], add=True)
            if i >= 2:
                psl = (i - 2) % NSLOT
                pltpu.make_async_copy(
                    rows[psl], agg.at[dstb.at[i - 2]], ss[psl]).wait()
            if i + 2 < CB:
                nsl = (i + 2) % NSLOT
                pltpu.async_copy(
                    xw_hbm.at[srcb.at[pl.ds((i + 2) * K, K)]], rows[nsl],
                    sg[nsl])
        for i in range(CB - 2, CB):
            sl = i % NSLOT
            pltpu.make_async_copy(
                rows[sl], agg.at[dstb.at[i]], ss[sl]).wait()

    plsc.subcore_barrier()
    pltpu.sync_copy(agg.at[pl.ds(s * ROWS_PER_TILE, ROWS_PER_TILE)],
                    out_hbm.at[c, pl.ds(s * ROWS_PER_TILE, ROWS_PER_TILE)])


def _sc_agg(ef, et, xw_flat):
    mesh = plsc.VectorSubcoreMesh(core_axis_name="c", subcore_axis_name="s")
    return pl.kernel(
        _sc_agg_body,
        out_type=jax.ShapeDtypeStruct((NC, NPAD, D), jnp.float32),
        mesh=mesh,
        scratch_types=[
            pltpu.VMEM((CB * K,), jnp.int32),
            pltpu.VMEM((CB * K,), jnp.int32),
            pltpu.VMEM((CB * K,), jnp.int32),
            pltpu.VMEM((CB, K), jnp.int32),
            pltpu.VMEM((K, D), jnp.float32),
            pltpu.VMEM((K, D), jnp.float32),
            pltpu.VMEM((K, D), jnp.float32),
            pltpu.VMEM((K, D), jnp.float32),
            pltpu.VMEM_SHARED((NPAD, D), jnp.float32),
            pltpu.SemaphoreType.DMA,
            pltpu.SemaphoreType.DMA,
            pltpu.SemaphoreType.DMA,
            pltpu.SemaphoreType.DMA,
            pltpu.SemaphoreType.DMA,
            pltpu.SemaphoreType.DMA,
            pltpu.SemaphoreType.DMA,
            pltpu.SemaphoreType.DMA,
        ],
    )(ef, et, xw_flat)


def _tc_curr_body(x_ref, w_ref, b_ref, out_ref):
    acc = jnp.dot(x_ref[...], w_ref[...], preferred_element_type=jnp.float32)
    out_ref[...] = acc + b_ref[...]


def _tc_curr(x, self_loop_weight, bias):
    blk = 1000
    return pl.pallas_call(
        _tc_curr_body,
        grid=(N // blk,),
        in_specs=[
            pl.BlockSpec((blk, D), lambda i: (i, 0)),
            pl.BlockSpec((D, D), lambda i: (0, 0)),
            pl.BlockSpec((1, D), lambda i: (0, 0)),
        ],
        out_specs=pl.BlockSpec((blk, D), lambda i: (i, 0)),
        out_shape=jax.ShapeDtypeStruct((N, D), jnp.float32),
    )(x, self_loop_weight, bias.reshape(1, D))


def _tc_out_body(curr_ref, agg_ref, out_ref):
    out_ref[...] = jnp.maximum(curr_ref[...] + agg_ref[0] + agg_ref[1], 0.0)


def _tc_out(curr, agg_pair):
    blk = 1000
    return pl.pallas_call(
        _tc_out_body,
        grid=(N // blk,),
        in_specs=[
            pl.BlockSpec((blk, D), lambda i: (i, 0)),
            pl.BlockSpec((NC, blk, D), lambda i: (0, i, 0)),
        ],
        out_specs=pl.BlockSpec((blk, D), lambda i: (i, 0)),
        out_shape=jax.ShapeDtypeStruct((N, D), jnp.float32),
    )(curr, agg_pair)


def kernel(x, edge_index, edge_type, weight_bases, w_comp, self_loop_weight, bias):
    ef = edge_index.astype(jnp.int32).reshape(2 * E)
    et = edge_type.astype(jnp.int32)

    xw = _tc_xw(x, w_comp, weight_bases)
    curr = _tc_curr(x, self_loop_weight, bias)
    agg_pair = _sc_agg(ef, et, xw.reshape(R * N, D))
    return _tc_out(curr, agg_pair)


# speedup vs baseline: 1.0787x; 1.0014x over previous
import jax
import jax.numpy as jnp
from jax import lax
from jax.experimental import pallas as pl
from jax.experimental.pallas import tpu as pltpu
from jax.experimental.pallas import tpu_sc as plsc

N = 10000
E = 320000
D = 128
R = 32
NB = 8

NC = 2
NS = 16

K = 64
CB = 40
BPW = 160
NCHUNK = BPW // CB
NROWS = E // K
NPAD = 10112
ROWS_PER_TILE = NPAD // NS
NSLOT = 4


def _tc_xw_body(x_ref, wc_ref, bases_ref, out_ref):
    w = wc_ref[0, 0, 0] * bases_ref[0]
    for b in range(1, NB):
        w += wc_ref[0, 0, b] * bases_ref[b]
    out_ref[0] = jnp.dot(x_ref[...], w, preferred_element_type=jnp.float32)


def _tc_xw(x, w_comp, weight_bases):
    return pl.pallas_call(
        _tc_xw_body,
        grid=(R,),
        in_specs=[
            pl.BlockSpec((N, D), lambda r: (0, 0)),
            pl.BlockSpec((1, 1, NB), lambda r: (r, 0, 0)),
            pl.BlockSpec((NB, D, D), lambda r: (0, 0, 0)),
        ],
        out_specs=pl.BlockSpec((1, N, D), lambda r: (r, 0, 0)),
        out_shape=jax.ShapeDtypeStruct((R, N, D), jnp.float32),
    )(x, w_comp.reshape(R, 1, NB), weight_bases)


def _sc_agg_body(ef_hbm, et_hbm, xw_hbm, out_hbm,
                 srcb, relb, dstl, dstb,
                 rows0, rows1, rows2, rows3, agg,
                 sg0, sg1, sg2, sg3, ss0, ss1, ss2, ss3):
    c = lax.axis_index("c")
    s = lax.axis_index("s")

    @pl.loop(0, K)
    def _(i):
        for j in range(D // 16):
            rows0[i, pl.ds(j * 16, 16)] = jnp.zeros((16,), jnp.float32)

    for k in range(ROWS_PER_TILE // K + 1):
        cnt = K if k < ROWS_PER_TILE // K else ROWS_PER_TILE % K
        pltpu.sync_copy(rows0.at[pl.ds(0, cnt)],
                        agg.at[pl.ds(s * ROWS_PER_TILE + k * K, cnt)])
    plsc.subcore_barrier()

    rows = (rows0, rows1, rows2, rows3)
    sg = (sg0, sg1, sg2, sg3)
    ss = (ss0, ss1, ss2, ss3)
    wid = s * NC + c
    nchunk = lax.select(wid == NC * NS - 1, 1, NCHUNK)

    @pl.loop(0, nchunk)
    def _(cc):
        ebase = (wid * BPW + cc * CB) * K
        esl = pl.ds(ebase, CB * K)
        pltpu.sync_copy(ef_hbm.at[esl], srcb)
        pltpu.sync_copy(et_hbm.at[esl], relb)
        pltpu.sync_copy(ef_hbm.at[pl.ds(E + ebase, CB * K)], dstl)

        for i in range(CB):
            for j in range(K // 16):
                sl = pl.ds(j * 16, 16)
                fl = pl.ds(i * K + j * 16, 16)
                srcb[fl] = relb[fl] * N + srcb[fl]
                dstb[i, sl] = dstl[fl]

        pltpu.async_copy(xw_hbm.at[srcb.at[pl.ds(0, K)]], rows[0], sg[0])
        pltpu.async_copy(xw_hbm.at[srcb.at[pl.ds(K, K)]], rows[1], sg[1])
        for i in range(CB):
            sl = i % NSLOT
            pltpu.make_async_copy(
                xw_hbm.at[srcb.at[pl.ds(i * K, K)]], rows[sl], sg[sl]).wait()
            pltpu.async_copy(rows[sl], agg.at[dstb.at[i]], ss[sl], add=True)
            if i >= 2:
                psl = (i - 2) % NSLOT
                pltpu.make_async_copy(
                    rows[psl], agg.at[dstb.at[i - 2]], ss[psl]).wait()
            if i + 2 < CB:
                nsl = (i + 2) % NSLOT
                pltpu.async_copy(
                    xw_hbm.at[srcb.at[pl.ds((i + 2) * K, K)]], rows[nsl],
                    sg[nsl])
        for i in range(CB - 2, CB):
            sl = i % NSLOT
            pltpu.make_async_copy(
                rows[sl], agg.at[dstb.at[i]], ss[sl]).wait()

    plsc.subcore_barrier()
    pltpu.sync_copy(agg.at[pl.ds(s * ROWS_PER_TILE, ROWS_PER_TILE)],
                    out_hbm.at[c, pl.ds(s * ROWS_PER_TILE, ROWS_PER_TILE)])


def _sc_agg(ef, et, xw_flat):
    mesh = plsc.VectorSubcoreMesh(core_axis_name="c", subcore_axis_name="s")
    return pl.kernel(
        _sc_agg_body,
        out_type=jax.ShapeDtypeStruct((NC, NPAD, D), jnp.float32),
        mesh=mesh,
        scratch_types=[
            pltpu.VMEM((CB * K,), jnp.int32),
            pltpu.VMEM((CB * K,), jnp.int32),
            pltpu.VMEM((CB * K,), jnp.int32),
            pltpu.VMEM((CB, K), jnp.int32),
            pltpu.VMEM((K, D), jnp.float32),
            pltpu.VMEM((K, D), jnp.float32),
            pltpu.VMEM((K, D), jnp.float32),
            pltpu.VMEM((K, D), jnp.float32),
            pltpu.VMEM_SHARED((NPAD, D), jnp.float32),
            pltpu.SemaphoreType.DMA,
            pltpu.SemaphoreType.DMA,
            pltpu.SemaphoreType.DMA,
            pltpu.SemaphoreType.DMA,
            pltpu.SemaphoreType.DMA,
            pltpu.SemaphoreType.DMA,
            pltpu.SemaphoreType.DMA,
            pltpu.SemaphoreType.DMA,
        ],
    )(ef, et, xw_flat)


def _tc_curr_body(x_ref, w_ref, b_ref, out_ref):
    acc = jnp.dot(x_ref[...], w_ref[...], preferred_element_type=jnp.float32)
    out_ref[...] = acc + b_ref[...]


def _tc_curr(x, self_loop_weight, bias):
    blk = 1000
    return pl.pallas_call(
        _tc_curr_body,
        grid=(N // blk,),
        in_specs=[
            pl.BlockSpec((blk, D), lambda i: (i, 0)),
            pl.BlockSpec((D, D), lambda i: (0, 0)),
            pl.BlockSpec((1, D), lambda i: (0, 0)),
        ],
        out_specs=pl.BlockSpec((blk, D), lambda i: (i, 0)),
        out_shape=jax.ShapeDtypeStruct((N, D), jnp.float32),
    )(x, self_loop_weight, bias.reshape(1, D))


def _tc_out_body(curr_ref, agg_ref, out_ref):
    out_ref[...] = jnp.maximum(curr_ref[...] + agg_ref[0] + agg_ref[1], 0.0)


def _tc_out(curr, agg_pair):
    blk = 1000
    return pl.pallas_call(
        _tc_out_body,
        grid=(N // blk,),
        in_specs=[
            pl.BlockSpec((blk, D), lambda i: (i, 0)),
            pl.BlockSpec((NC, blk, D), lambda i: (0, i, 0)),
        ],
        out_specs=pl.BlockSpec((blk, D), lambda i: (i, 0)),
        out_shape=jax.ShapeDtypeStruct((N, D), jnp.float32),
    )(curr, agg_pair)


def kernel(x, edge_index, edge_type, weight_bases, w_comp, self_loop_weight, bias):
    ef = edge_index.astype(jnp.int32).reshape(2 * E)
    et = edge_type.astype(jnp.int32)

    xw = _tc_xw(x, w_comp, weight_bases)
    curr = _tc_curr(x, self_loop_weight, bias)
    agg_pair = _sc_agg(ef, et, xw.reshape(R * N, D))
    return _tc_out(curr, agg_pair)
